# baseline (speedup 1.0000x reference)
import functools

import jax
import jax.numpy as jnp
from jax import lax
from jax.experimental import pallas as pl
from jax.experimental.pallas import tpu as pltpu
from jax.experimental.pallas import tpu_sc as plsc

EMB_D = 32
NUM_CORES = 2
NUM_SUBCORES = 16
NW = NUM_CORES * NUM_SUBCORES
CHUNK = 128


@functools.lru_cache(maxsize=None)
def _make_gather(B: int, V: int):
    assert B % (NW * CHUNK) == 0
    b_per_w = B // NW
    n_chunks = b_per_w // CHUNK
    mesh = plsc.VectorSubcoreMesh(core_axis_name="c", subcore_axis_name="s")

    @functools.partial(
        pl.kernel,
        mesh=mesh,
        out_type=jax.ShapeDtypeStruct((B, EMB_D), jnp.float32),
        scratch_types=[
            pltpu.VMEM((CHUNK,), jnp.int32),
            pltpu.VMEM((CHUNK, EMB_D), jnp.float32),
            pltpu.SemaphoreType.DMA,
        ],
        compiler_params=pltpu.CompilerParams(use_tc_tiling_on_sc=False),
    )
    def gather_kernel(idx_hbm, table_hbm, out_hbm, idx_v, rows_v, sem):
        wid = lax.axis_index("s") * NUM_CORES + lax.axis_index("c")
        base = wid * b_per_w

        def body(g, carry):
            off = base + g * CHUNK
            pltpu.sync_copy(idx_hbm.at[pl.ds(off, CHUNK)], idx_v)
            pltpu.async_copy(table_hbm.at[idx_v], rows_v, sem).wait()
            pltpu.sync_copy(rows_v, out_hbm.at[pl.ds(off, CHUNK)])
            return carry

        lax.fori_loop(0, n_chunks, body, 0)

    return gather_kernel


def kernel(x, W):
    rows, cols = x.shape
    B = rows * cols
    xf = x.reshape(B).astype(jnp.int32)
    out = _make_gather(B, W.shape[0])(xf, W)
    return out.reshape(rows, cols, EMB_D)


# speedup vs baseline: 1.1314x; 1.1314x over previous
import functools

import jax
import jax.numpy as jnp
from jax import lax
from jax.experimental import pallas as pl
from jax.experimental.pallas import tpu as pltpu
from jax.experimental.pallas import tpu_sc as plsc

EMB_D = 32
NUM_CORES = 2
NUM_SUBCORES = 16
NW = NUM_CORES * NUM_SUBCORES
CHUNK = 128
K = 5
SUP = K * CHUNK


@functools.lru_cache(maxsize=None)
def _make_gather(B: int):
    b_per_w = B // NW
    n_chunks = b_per_w // CHUNK
    n_super = b_per_w // SUP
    assert b_per_w % SUP == 0 and n_super % 2 == 0
    n_iter = n_super // 2
    mesh = plsc.VectorSubcoreMesh(core_axis_name="c", subcore_axis_name="s")

    @functools.partial(
        pl.kernel,
        mesh=mesh,
        out_type=jax.ShapeDtypeStruct((B, EMB_D), jnp.float32),
        scratch_types=[
            pltpu.VMEM((n_chunks, CHUNK), jnp.int32),
            pltpu.VMEM((SUP, EMB_D), jnp.float32),
            pltpu.VMEM((SUP, EMB_D), jnp.float32),
            pltpu.SemaphoreType.DMA,
            pltpu.SemaphoreType.DMA,
            pltpu.SemaphoreType.DMA,
        ],
        compiler_params=pltpu.CompilerParams(use_tc_tiling_on_sc=False),
    )
    def gather_kernel(idx_hbm, table_hbm, out_hbm, idx_v, rows_a, rows_b,
                      gsem, wsem_a, wsem_b):
        wid = lax.axis_index("s") * NUM_CORES + lax.axis_index("c")
        base = wid * b_per_w

        pltpu.sync_copy(idx_hbm.at[pl.ds(wid * n_chunks, n_chunks)], idx_v)

        def fire_gathers(s, buf):
            for b in range(K):
                pltpu.async_copy(
                    table_hbm.at[idx_v.at[s * K + b]],
                    buf.at[pl.ds(b * CHUNK, CHUNK)],
                    gsem,
                )

        def drain_gathers(buf):
            pltpu.make_async_copy(table_hbm.at[pl.ds(0, SUP)], buf, gsem).wait()

        def fire_write(s, buf, wsem):
            pltpu.async_copy(buf, out_hbm.at[pl.ds(base + s * SUP, SUP)], wsem)

        def drain_write(buf, wsem):
            pltpu.make_async_copy(buf, out_hbm.at[pl.ds(0, SUP)], wsem).wait()

        fire_gathers(0, rows_a)

        def body(t, carry):
            s0 = 2 * t
            s1 = s0 + 1
            drain_gathers(rows_a)
            fire_write(s0, rows_a, wsem_a)

            @pl.when(t > 0)
            def _():
                drain_write(rows_b, wsem_b)

            fire_gathers(s1, rows_b)
            drain_gathers(rows_b)
            fire_write(s1, rows_b, wsem_b)
            drain_write(rows_a, wsem_a)

            @pl.when(t < n_iter - 1)
            def _():
                fire_gathers(s0 + 2, rows_a)

            return carry

        lax.fori_loop(0, n_iter, body, 0)
        drain_write(rows_b, wsem_b)

    return gather_kernel


def kernel(x, W):
    rows, cols = x.shape
    B = rows * cols
    xf = x.reshape(B // CHUNK, CHUNK).astype(jnp.int32)
    out = _make_gather(B)(xf, W)
    return out.reshape(rows, cols, EMB_D)
